# Initial kernel scaffold; baseline (speedup 1.0000x reference)
import jax
import jax.numpy as jnp
from jax.experimental import pallas as pl


def kernel(QR, symbols_weight):
    raise NotImplementedError("write your pallas kernel here")


# speedup vs baseline: 1.8513x; 1.8513x over previous
import functools

import jax
import jax.numpy as jnp
from jax import lax
from jax.experimental import pallas as pl
from jax.experimental.pallas import tpu as pltpu
from jax.experimental.pallas import tpu_sc as plsc

DIM = 128
CH = 128


def _make_sc_gather(N, n_workers, n_per_w, n_ch):
    mesh = plsc.VectorSubcoreMesh(core_axis_name="c", subcore_axis_name="s")
    nc = mesh.num_cores

    @functools.partial(
        pl.kernel,
        out_type=jax.ShapeDtypeStruct((N, DIM), jnp.float32),
        mesh=mesh,
        scratch_types=[
            pltpu.VMEM((n_ch, CH), jnp.int32),
            pltpu.VMEM((CH, DIM), jnp.float32),
            pltpu.SemaphoreType.DMA,
        ],
    )
    def sc_gather(idx_hbm, table_hbm, out_hbm, idx_v, rows_v, gsem):
        wid = lax.axis_index("s") * nc + lax.axis_index("c")
        pltpu.sync_copy(idx_hbm.at[pl.ds(wid * n_ch, n_ch), :], idx_v)
        base = wid * n_per_w

        def chunk(j, carry):
            pltpu.async_copy(table_hbm.at[idx_v.at[j]], rows_v, gsem).wait()
            pltpu.sync_copy(rows_v, out_hbm.at[pl.ds(base + j * CH, CH), :])
            return carry

        lax.fori_loop(0, n_ch, chunk, 0)

    return sc_gather


def kernel(QR, symbols_weight):
    B, T = QR.shape
    N = B * T
    n_workers = 32
    n_per_w = N // n_workers
    n_ch = n_per_w // CH
    idx = QR.reshape(n_workers * n_ch, CH).astype(jnp.int32)
    out = _make_sc_gather(N, n_workers, n_per_w, n_ch)(idx, symbols_weight)
    return out.reshape(B, T, DIM)
